# baseline (speedup 1.0000x reference)
import functools

import jax
import jax.numpy as jnp
from jax import lax
from jax.experimental import pallas as pl
from jax.experimental.pallas import tpu as pltpu
from jax.experimental.pallas import tpu_sc as plsc

CHUNK = 128
NCORES = 2
NSUB = 16
NTILES = NCORES * NSUB
NBUF = 4


def _mesh():
    return plsc.VectorSubcoreMesh(core_axis_name="c", subcore_axis_name="s")


def _deg_kernel(n_pad, nch, dw):
    rpt = n_pad // NSUB

    @functools.partial(
        pl.kernel,
        out_type=jax.ShapeDtypeStruct((NCORES, n_pad, dw), jnp.float32),
        mesh=_mesh(),
        compiler_params=pltpu.CompilerParams(use_tc_tiling_on_sc=False),
        scratch_types=[
            pltpu.VMEM((nch, CHUNK), jnp.int32),
            pltpu.VMEM((CHUNK, dw), jnp.float32),
            pltpu.VMEM_SHARED((n_pad, dw), jnp.float32),
            pltpu.SemaphoreType.DMA,
        ],
    )
    def k(dst_hbm, zeros_hbm, ones_hbm, out_hbm, dst_v, ones_v, acc, ssem):
        c = lax.axis_index("c")
        s = lax.axis_index("s")
        wid = s * NCORES + c
        pltpu.sync_copy(dst_hbm.at[pl.ds(wid * nch, nch)], dst_v)
        pltpu.sync_copy(ones_hbm, ones_v)
        pltpu.sync_copy(zeros_hbm.at[pl.ds(s * rpt, rpt)], acc.at[pl.ds(s * rpt, rpt)])
        plsc.subcore_barrier()

        def body(j, carry):
            pltpu.async_copy(ones_v, acc.at[dst_v.at[j]], ssem, add=True)
            return carry

        lax.fori_loop(0, nch, body, 0)

        def drain(j, carry):
            pltpu.make_async_copy(ones_v, acc.at[dst_v.at[j]], ssem).wait()
            return carry

        lax.fori_loop(0, nch, drain, 0)
        plsc.subcore_barrier()
        pltpu.sync_copy(acc.at[pl.ds(s * rpt, rpt)], out_hbm.at[c, pl.ds(s * rpt, rpt)])

    return k


def _scatter_kernel(n_pad, nch, f):
    rpt = n_pad // NSUB

    @functools.partial(
        pl.kernel,
        out_type=jax.ShapeDtypeStruct((NCORES, n_pad, f), jnp.float32),
        mesh=_mesh(),
        compiler_params=pltpu.CompilerParams(use_tc_tiling_on_sc=False),
        scratch_types=[
            pltpu.VMEM((nch, CHUNK), jnp.int32),
            pltpu.VMEM((nch, CHUNK), jnp.int32),
            [pltpu.VMEM((CHUNK, f), jnp.float32) for _ in range(NBUF)],
            [pltpu.SemaphoreType.DMA for _ in range(NBUF)],
            [pltpu.SemaphoreType.DMA for _ in range(NBUF)],
            pltpu.VMEM_SHARED((n_pad, f), jnp.float32),
        ],
    )
    def k(src_hbm, dst_hbm, y_hbm, zeros_hbm, out_hbm,
          src_v, dst_v, bufs, gsems, ssems, acc):
        c = lax.axis_index("c")
        s = lax.axis_index("s")
        wid = s * NCORES + c
        pltpu.sync_copy(src_hbm.at[pl.ds(wid * nch, nch)], src_v)
        pltpu.sync_copy(dst_hbm.at[pl.ds(wid * nch, nch)], dst_v)
        pltpu.sync_copy(zeros_hbm.at[pl.ds(s * rpt, rpt)], acc.at[pl.ds(s * rpt, rpt)])
        plsc.subcore_barrier()

        def gather(j, b):
            pltpu.async_copy(y_hbm.at[src_v.at[j]], bufs[b], gsems[b])

        def gather_wait(j, b):
            pltpu.make_async_copy(y_hbm.at[src_v.at[j]], bufs[b], gsems[b]).wait()

        def scat(j, b):
            pltpu.async_copy(bufs[b], acc.at[dst_v.at[j]], ssems[b], add=True)

        def scat_wait(j, b):
            pltpu.make_async_copy(bufs[b], acc.at[dst_v.at[j]], ssems[b]).wait()

        for b in range(NBUF):
            gather(b, b)

        def body(i, carry):
            j0 = NBUF * i
            for b in range(NBUF):
                gather_wait(j0 + b, b)
                scat(j0 + b, b)
            for b in range(NBUF):
                scat_wait(j0 + b, b)
                gather(j0 + NBUF + b, b)
            return carry

        lax.fori_loop(0, nch // NBUF - 1, body, 0)
        j0 = nch - NBUF
        for b in range(NBUF):
            gather_wait(j0 + b, b)
            scat(j0 + b, b)
        for b in range(NBUF):
            scat_wait(j0 + b, b)

        plsc.subcore_barrier()
        pltpu.sync_copy(acc.at[pl.ds(s * rpt, rpt)], out_hbm.at[c, pl.ds(s * rpt, rpt)])

    return k


def _tc_prep(deg2, x, w1, n):
    h = w1.shape[1]

    def body(deg_ref, x_ref, w_ref, y_ref, dinv_ref):
        d = deg_ref[0, :n, 0:1] + deg_ref[1, :n, 0:1] + 1.0
        dinv = lax.rsqrt(d)
        xt = jnp.dot(x_ref[...], w_ref[...], preferred_element_type=jnp.float32)
        y_ref[...] = xt * dinv
        dinv_ref[...] = dinv

    return pl.pallas_call(
        body,
        out_shape=[
            jax.ShapeDtypeStruct((n, h), jnp.float32),
            jax.ShapeDtypeStruct((n, 1), jnp.float32),
        ],
    )(deg2, x, w1)


def _tc_mid(s1, y1, dinv, b1, w2p, n):
    w = w2p.shape[1]

    def body(s_ref, y_ref, dinv_ref, b_ref, w_ref, o_ref):
        o1 = (s_ref[0, :n, :] + s_ref[1, :n, :] + y_ref[...]) * dinv_ref[...] + b_ref[...]
        hid = jnp.maximum(o1, 0.0)
        o_ref[...] = jnp.dot(hid, w_ref[...], preferred_element_type=jnp.float32) * dinv_ref[...]

    return pl.pallas_call(
        body,
        out_shape=jax.ShapeDtypeStruct((n, w), jnp.float32),
    )(s1, y1, dinv, b1, w2p)


def _tc_final(s2, y2, dinv, b2, n, c_out):
    def body(s_ref, y_ref, dinv_ref, b_ref, o_ref):
        o = (s_ref[0, :n, :] + s_ref[1, :n, :] + y_ref[...]) * dinv_ref[...]
        logits = o[:, 0:c_out] + b_ref[...]
        m = jnp.max(logits, axis=1, keepdims=True)
        ex = jnp.exp(logits - m)
        lse = jnp.log(jnp.sum(ex, axis=1, keepdims=True))
        o_ref[...] = logits - m - lse

    return pl.pallas_call(
        body,
        out_shape=jax.ShapeDtypeStruct((n, c_out), jnp.float32),
    )(s2, y2, dinv, b2)


def kernel(x, edge_index, W1, b1, W2, b2):
    n, _ = x.shape
    h = W1.shape[1]
    c_out = W2.shape[1]
    e = edge_index.shape[1]

    nch = -(-e // (NTILES * CHUNK))
    nch = -(-nch // NBUF) * NBUF
    e_pad = NTILES * CHUNK * nch
    n_pad = -(-(n + 1) // (NSUB * 8)) * (NSUB * 8)
    dw = 16
    w2w = 16

    src = jnp.concatenate(
        [edge_index[0], jnp.zeros((e_pad - e,), jnp.int32)]).reshape(-1, CHUNK)
    dst = jnp.concatenate(
        [edge_index[1], jnp.full((e_pad - e,), n, jnp.int32)]).reshape(-1, CHUNK)

    zeros_dw = jnp.zeros((n_pad, dw), jnp.float32)
    ones_dw = jnp.ones((CHUNK, dw), jnp.float32)
    zeros_h = jnp.zeros((n_pad, h), jnp.float32)
    zeros_w2 = jnp.zeros((n_pad, w2w), jnp.float32)
    w2p = jnp.pad(W2, ((0, 0), (0, w2w - c_out)))

    deg2 = _deg_kernel(n_pad, nch, dw)(dst, zeros_dw, ones_dw)
    y1, dinv = _tc_prep(deg2, x, W1, n)
    s1 = _scatter_kernel(n_pad, nch, h)(src, dst, y1, zeros_h)
    y2 = _tc_mid(s1, y1, dinv, b1.reshape(1, h), w2p, n)
    s2 = _scatter_kernel(n_pad, nch, w2w)(src, dst, y2, zeros_w2)
    return _tc_final(s2, y2, dinv, b2.reshape(1, c_out), n, c_out)


# speedup vs baseline: 1.5564x; 1.5564x over previous
import functools

import jax
import jax.numpy as jnp
from jax import lax
from jax.experimental import pallas as pl
from jax.experimental.pallas import tpu as pltpu
from jax.experimental.pallas import tpu_sc as plsc

CHUNK = 128
NCORES = 2
NSUB = 16
NTILES = NCORES * NSUB
NBUF = 4


def _mesh():
    return plsc.VectorSubcoreMesh(core_axis_name="c", subcore_axis_name="s")


def _deg_kernel(n_pad, nch, dw):
    rpt = n_pad // NSUB

    @functools.partial(
        pl.kernel,
        out_type=jax.ShapeDtypeStruct((NCORES, n_pad, dw), jnp.float32),
        mesh=_mesh(),
        compiler_params=pltpu.CompilerParams(use_tc_tiling_on_sc=False),
        scratch_types=[
            pltpu.VMEM((nch, CHUNK), jnp.int32),
            pltpu.VMEM((CHUNK, dw), jnp.float32),
            pltpu.VMEM_SHARED((n_pad, dw), jnp.float32),
            pltpu.SemaphoreType.DMA,
        ],
    )
    def k(dst_hbm, zeros_hbm, ones_hbm, out_hbm, dst_v, ones_v, acc, ssem):
        c = lax.axis_index("c")
        s = lax.axis_index("s")
        wid = s * NCORES + c
        pltpu.sync_copy(dst_hbm.at[pl.ds(wid * nch, nch)], dst_v)
        pltpu.sync_copy(ones_hbm, ones_v)
        pltpu.sync_copy(zeros_hbm.at[pl.ds(s * rpt, rpt)], acc.at[pl.ds(s * rpt, rpt)])
        plsc.subcore_barrier()

        def body(j, carry):
            pltpu.async_copy(ones_v, acc.at[dst_v.at[j]], ssem, add=True)
            return carry

        lax.fori_loop(0, nch, body, 0)

        def drain(j, carry):
            pltpu.make_async_copy(ones_v, acc.at[dst_v.at[j]], ssem).wait()
            return carry

        lax.fori_loop(0, nch, drain, 0)
        plsc.subcore_barrier()
        pltpu.sync_copy(acc.at[pl.ds(s * rpt, rpt)], out_hbm.at[c, pl.ds(s * rpt, rpt)])

    return k


def _scatter_kernel(n_pad, nch, f):
    rpt = n_pad // NSUB

    @functools.partial(
        pl.kernel,
        out_type=jax.ShapeDtypeStruct((NCORES, n_pad, f), jnp.float32),
        mesh=_mesh(),
        compiler_params=pltpu.CompilerParams(use_tc_tiling_on_sc=False),
        scratch_types=[
            pltpu.VMEM((nch, CHUNK), jnp.int32),
            pltpu.VMEM((nch, CHUNK), jnp.int32),
            [pltpu.VMEM((CHUNK, f), jnp.float32) for _ in range(NBUF)],
            [pltpu.SemaphoreType.DMA for _ in range(NBUF)],
            [pltpu.SemaphoreType.DMA for _ in range(NBUF)],
            pltpu.VMEM_SHARED((n_pad, f), jnp.float32),
            pltpu.VMEM_SHARED((n_pad, f), jnp.float32),
        ],
    )
    def k(src_hbm, dst_hbm, y_hbm, zeros_hbm, out_hbm,
          src_v, dst_v, bufs, gsems, ssems, acc, y_sp):
        c = lax.axis_index("c")
        s = lax.axis_index("s")
        wid = s * NCORES + c
        pltpu.sync_copy(src_hbm.at[pl.ds(wid * nch, nch)], src_v)
        pltpu.sync_copy(dst_hbm.at[pl.ds(wid * nch, nch)], dst_v)
        pltpu.sync_copy(zeros_hbm.at[pl.ds(s * rpt, rpt)], acc.at[pl.ds(s * rpt, rpt)])
        pltpu.sync_copy(y_hbm.at[pl.ds(s * rpt, rpt)], y_sp.at[pl.ds(s * rpt, rpt)])
        plsc.subcore_barrier()

        def gather(j, b):
            pltpu.async_copy(y_sp.at[src_v.at[j]], bufs[b], gsems[b])

        def gather_wait(j, b):
            pltpu.make_async_copy(y_sp.at[src_v.at[j]], bufs[b], gsems[b]).wait()

        def scat(j, b):
            pltpu.async_copy(bufs[b], acc.at[dst_v.at[j]], ssems[b], add=True)

        def scat_wait(j, b):
            pltpu.make_async_copy(bufs[b], acc.at[dst_v.at[j]], ssems[b]).wait()

        for b in range(NBUF):
            gather(b, b)

        def body(i, carry):
            j0 = NBUF * i
            for b in range(NBUF):
                gather_wait(j0 + b, b)
                scat(j0 + b, b)
            for b in range(NBUF):
                scat_wait(j0 + b, b)
                gather(j0 + NBUF + b, b)
            return carry

        lax.fori_loop(0, nch // NBUF - 1, body, 0)
        j0 = nch - NBUF
        for b in range(NBUF):
            gather_wait(j0 + b, b)
            scat(j0 + b, b)
        for b in range(NBUF):
            scat_wait(j0 + b, b)

        plsc.subcore_barrier()
        pltpu.sync_copy(acc.at[pl.ds(s * rpt, rpt)], out_hbm.at[c, pl.ds(s * rpt, rpt)])

    return k


def _tc_prep(deg2, x, w1, n):
    h = w1.shape[1]

    def body(deg_ref, x_ref, w_ref, y_ref, dinv_ref):
        d = deg_ref[0, :n, 0:1] + deg_ref[1, :n, 0:1] + 1.0
        dinv = lax.rsqrt(d)
        xt = jnp.dot(x_ref[...], w_ref[...], preferred_element_type=jnp.float32)
        y_ref[...] = xt * dinv
        dinv_ref[...] = dinv

    return pl.pallas_call(
        body,
        out_shape=[
            jax.ShapeDtypeStruct((n, h), jnp.float32),
            jax.ShapeDtypeStruct((n, 1), jnp.float32),
        ],
    )(deg2, x, w1)


def _tc_mid(s1, y1, dinv, b1, w2p, n):
    w = w2p.shape[1]

    def body(s_ref, y_ref, dinv_ref, b_ref, w_ref, o_ref):
        o1 = (s_ref[0, :n, :] + s_ref[1, :n, :] + y_ref[...]) * dinv_ref[...] + b_ref[...]
        hid = jnp.maximum(o1, 0.0)
        o_ref[...] = jnp.dot(hid, w_ref[...], preferred_element_type=jnp.float32) * dinv_ref[...]

    return pl.pallas_call(
        body,
        out_shape=jax.ShapeDtypeStruct((n, w), jnp.float32),
    )(s1, y1, dinv, b1, w2p)


def _tc_final(s2, y2, dinv, b2, n, c_out):
    def body(s_ref, y_ref, dinv_ref, b_ref, o_ref):
        o = (s_ref[0, :n, :] + s_ref[1, :n, :] + y_ref[...]) * dinv_ref[...]
        logits = o[:, 0:c_out] + b_ref[...]
        m = jnp.max(logits, axis=1, keepdims=True)
        ex = jnp.exp(logits - m)
        lse = jnp.log(jnp.sum(ex, axis=1, keepdims=True))
        o_ref[...] = logits - m - lse

    return pl.pallas_call(
        body,
        out_shape=jax.ShapeDtypeStruct((n, c_out), jnp.float32),
    )(s2, y2, dinv, b2)


def kernel(x, edge_index, W1, b1, W2, b2):
    n, _ = x.shape
    h = W1.shape[1]
    c_out = W2.shape[1]
    e = edge_index.shape[1]

    nch = -(-e // (NTILES * CHUNK))
    nch = -(-nch // NBUF) * NBUF
    e_pad = NTILES * CHUNK * nch
    n_pad = -(-(n + 1) // (NSUB * 8)) * (NSUB * 8)
    dw = 16
    w2w = 16

    src = jnp.concatenate(
        [edge_index[0], jnp.zeros((e_pad - e,), jnp.int32)]).reshape(-1, CHUNK)
    dst = jnp.concatenate(
        [edge_index[1], jnp.full((e_pad - e,), n, jnp.int32)]).reshape(-1, CHUNK)

    zeros_dw = jnp.zeros((n_pad, dw), jnp.float32)
    ones_dw = jnp.ones((CHUNK, dw), jnp.float32)
    zeros_h = jnp.zeros((n_pad, h), jnp.float32)
    zeros_w2 = jnp.zeros((n_pad, w2w), jnp.float32)
    w2p = jnp.pad(W2, ((0, 0), (0, w2w - c_out)))

    deg2 = _deg_kernel(n_pad, nch, dw)(dst, zeros_dw, ones_dw)
    y1, dinv = _tc_prep(deg2, x, W1, n)
    y1p = jnp.pad(y1, ((0, n_pad - n), (0, 0)))
    s1 = _scatter_kernel(n_pad, nch, h)(src, dst, y1p, zeros_h)
    y2 = _tc_mid(s1, y1, dinv, b1.reshape(1, h), w2p, n)
    y2p = jnp.pad(y2, ((0, n_pad - n), (0, 0)))
    s2 = _scatter_kernel(n_pad, nch, w2w)(src, dst, y2p, zeros_w2)
    return _tc_final(s2, y2, dinv, b2.reshape(1, c_out), n, c_out)


# speedup vs baseline: 1.6749x; 1.0761x over previous
import functools

import jax
import jax.numpy as jnp
from jax import lax
from jax.experimental import pallas as pl
from jax.experimental.pallas import tpu as pltpu
from jax.experimental.pallas import tpu_sc as plsc

CHUNK = 128
NCORES = 2
NSUB = 16
NTILES = NCORES * NSUB
NBUF = 4


def _mesh():
    return plsc.VectorSubcoreMesh(core_axis_name="c", subcore_axis_name="s")


def _tile_plan(wid, lo, t0):
    hi = lo + 4
    count = jnp.where(wid < t0, lo, hi)
    base = jnp.where(wid < t0, lo * wid, lo * t0 + hi * (wid - t0))
    return count, base


def _deg_kernel(n_pad, lo, t0, win, dw):
    rpt = n_pad // NSUB

    @functools.partial(
        pl.kernel,
        out_type=jax.ShapeDtypeStruct((NCORES, n_pad, dw), jnp.float32),
        mesh=_mesh(),
        compiler_params=pltpu.CompilerParams(use_tc_tiling_on_sc=False),
        scratch_types=[
            pltpu.VMEM((win, CHUNK), jnp.int32),
            pltpu.VMEM((CHUNK, dw), jnp.float32),
            pltpu.VMEM_SHARED((n_pad, dw), jnp.float32),
            pltpu.SemaphoreType.DMA,
        ],
    )
    def k(dst_hbm, zeros_hbm, ones_hbm, out_hbm, dst_v, ones_v, acc, ssem):
        c = lax.axis_index("c")
        s = lax.axis_index("s")
        wid = s * NCORES + c
        count, base = _tile_plan(wid, lo, t0)
        pltpu.sync_copy(dst_hbm.at[pl.ds(base, win)], dst_v)
        pltpu.sync_copy(ones_hbm, ones_v)
        pltpu.sync_copy(zeros_hbm.at[pl.ds(s * rpt, rpt)], acc.at[pl.ds(s * rpt, rpt)])
        plsc.subcore_barrier()

        def body(j, carry):
            pltpu.async_copy(ones_v, acc.at[dst_v.at[j]], ssem, add=True)
            return carry

        lax.fori_loop(0, count, body, 0)

        def drain(j, carry):
            pltpu.make_async_copy(ones_v, acc.at[dst_v.at[j]], ssem).wait()
            return carry

        lax.fori_loop(0, count, drain, 0)
        plsc.subcore_barrier()
        pltpu.sync_copy(acc.at[pl.ds(s * rpt, rpt)], out_hbm.at[c, pl.ds(s * rpt, rpt)])

    return k


def _scatter_kernel(n_pad, lo, t0, win, f):
    rpt = n_pad // NSUB

    @functools.partial(
        pl.kernel,
        out_type=jax.ShapeDtypeStruct((NCORES, n_pad, f), jnp.float32),
        mesh=_mesh(),
        compiler_params=pltpu.CompilerParams(use_tc_tiling_on_sc=False),
        scratch_types=[
            pltpu.VMEM((win, CHUNK), jnp.int32),
            pltpu.VMEM((win, CHUNK), jnp.int32),
            [pltpu.VMEM((CHUNK, f), jnp.float32) for _ in range(NBUF)],
            [pltpu.SemaphoreType.DMA for _ in range(NBUF)],
            [pltpu.SemaphoreType.DMA for _ in range(NBUF)],
            pltpu.VMEM_SHARED((n_pad, f), jnp.float32),
            pltpu.VMEM_SHARED((n_pad, f), jnp.float32),
        ],
    )
    def k(src_hbm, dst_hbm, y_hbm, zeros_hbm, out_hbm,
          src_v, dst_v, bufs, gsems, ssems, acc, y_sp):
        c = lax.axis_index("c")
        s = lax.axis_index("s")
        wid = s * NCORES + c
        count, base = _tile_plan(wid, lo, t0)
        pltpu.sync_copy(src_hbm.at[pl.ds(base, win)], src_v)
        pltpu.sync_copy(dst_hbm.at[pl.ds(base, win)], dst_v)
        pltpu.sync_copy(zeros_hbm.at[pl.ds(s * rpt, rpt)], acc.at[pl.ds(s * rpt, rpt)])
        pltpu.sync_copy(y_hbm.at[pl.ds(s * rpt, rpt)], y_sp.at[pl.ds(s * rpt, rpt)])
        plsc.subcore_barrier()

        def gather(j, b):
            pltpu.async_copy(y_sp.at[src_v.at[j]], bufs[b], gsems[b])

        def gather_wait(j, b):
            pltpu.make_async_copy(y_sp.at[src_v.at[j]], bufs[b], gsems[b]).wait()

        def scat(j, b):
            pltpu.async_copy(bufs[b], acc.at[dst_v.at[j]], ssems[b], add=True)

        def scat_wait(j, b):
            pltpu.make_async_copy(bufs[b], acc.at[dst_v.at[j]], ssems[b]).wait()

        for b in range(NBUF):
            gather(b, b)

        def body(i, carry):
            j0 = NBUF * i
            for b in range(NBUF):
                gather_wait(j0 + b, b)
                scat(j0 + b, b)
            for b in range(NBUF):
                scat_wait(j0 + b, b)
                gather(j0 + NBUF + b, b)
            return carry

        lax.fori_loop(0, count // NBUF - 1, body, 0)
        j0 = count - NBUF
        for b in range(NBUF):
            gather_wait(j0 + b, b)
            scat(j0 + b, b)
        for b in range(NBUF):
            scat_wait(j0 + b, b)

        plsc.subcore_barrier()
        pltpu.sync_copy(acc.at[pl.ds(s * rpt, rpt)], out_hbm.at[c, pl.ds(s * rpt, rpt)])

    return k


def _tc_prep(deg2, x, w1, n, n_pad):
    h = w1.shape[1]

    def body(deg_ref, x_ref, w_ref, y_ref, dinv_ref):
        d = deg_ref[0, :, 0:1] + deg_ref[1, :, 0:1] + 1.0
        dinv = lax.rsqrt(d)
        xt = jnp.dot(x_ref[...], w_ref[...], preferred_element_type=jnp.float32)
        xt_pad = jnp.concatenate(
            [xt, jnp.zeros((n_pad - n, h), jnp.float32)], axis=0)
        y_ref[...] = xt_pad * dinv
        dinv_ref[...] = dinv

    return pl.pallas_call(
        body,
        out_shape=[
            jax.ShapeDtypeStruct((n_pad, h), jnp.float32),
            jax.ShapeDtypeStruct((n_pad, 1), jnp.float32),
        ],
    )(deg2, x, w1)


def _tc_mid(s1, y1, dinv, b1, w2p, n_pad):
    w = w2p.shape[1]

    def body(s_ref, y_ref, dinv_ref, b_ref, w_ref, o_ref):
        o1 = (s_ref[0] + s_ref[1] + y_ref[...]) * dinv_ref[...] + b_ref[...]
        hid = jnp.maximum(o1, 0.0)
        o_ref[...] = jnp.dot(hid, w_ref[...], preferred_element_type=jnp.float32) * dinv_ref[...]

    return pl.pallas_call(
        body,
        out_shape=jax.ShapeDtypeStruct((n_pad, w), jnp.float32),
    )(s1, y1, dinv, b1, w2p)


def _tc_final(s2, y2, dinv, b2, n, c_out):
    def body(s_ref, y_ref, dinv_ref, b_ref, o_ref):
        o = (s_ref[0, :n, :] + s_ref[1, :n, :] + y_ref[:n, :]) * dinv_ref[:n, :]
        logits = o[:, 0:c_out] + b_ref[...]
        m = jnp.max(logits, axis=1, keepdims=True)
        ex = jnp.exp(logits - m)
        lse = jnp.log(jnp.sum(ex, axis=1, keepdims=True))
        o_ref[...] = logits - m - lse

    return pl.pallas_call(
        body,
        out_shape=jax.ShapeDtypeStruct((n, c_out), jnp.float32),
    )(s2, y2, dinv, b2)


def kernel(x, edge_index, W1, b1, W2, b2):
    n, _ = x.shape
    h = W1.shape[1]
    c_out = W2.shape[1]
    e = edge_index.shape[1]

    n_pad = -(-(n + 1) // (NSUB * 8)) * (NSUB * 8)
    dw = 16
    w2w = 16

    if e % (CHUNK * NBUF) == 0:
        src = edge_index[0].reshape(-1, CHUNK)
        dst = edge_index[1].reshape(-1, CHUNK)
    else:
        e_pad = -(-e // (CHUNK * NBUF)) * (CHUNK * NBUF)
        src = jnp.concatenate(
            [edge_index[0], jnp.zeros((e_pad - e,), jnp.int32)]).reshape(-1, CHUNK)
        dst = jnp.concatenate(
            [edge_index[1], jnp.full((e_pad - e,), n, jnp.int32)]).reshape(-1, CHUNK)
    n_chunks = src.shape[0]

    lo = (n_chunks // NTILES) // NBUF * NBUF
    n_hi = (n_chunks - NTILES * lo) // NBUF
    t0 = NTILES - n_hi
    win = lo + 4 if n_hi > 0 else lo

    zeros_dw = jnp.zeros((n_pad, dw), jnp.float32)
    ones_dw = jnp.ones((CHUNK, dw), jnp.float32)
    zeros_h = jnp.zeros((n_pad, h), jnp.float32)
    zeros_w2 = jnp.zeros((n_pad, w2w), jnp.float32)
    w2p = jnp.pad(W2, ((0, 0), (0, w2w - c_out)))

    deg2 = _deg_kernel(n_pad, lo, t0, win, dw)(dst, zeros_dw, ones_dw)
    y1, dinv = _tc_prep(deg2, x, W1, n, n_pad)
    s1 = _scatter_kernel(n_pad, lo, t0, win, h)(src, dst, y1, zeros_h)
    y2 = _tc_mid(s1, y1, dinv, b1.reshape(1, h), w2p, n_pad)
    s2 = _scatter_kernel(n_pad, lo, t0, win, w2w)(src, dst, y2, zeros_w2)
    return _tc_final(s2, y2, dinv, b2.reshape(1, c_out), n, c_out)


# speedup vs baseline: 1.7927x; 1.0703x over previous
import functools

import jax
import jax.numpy as jnp
from jax import lax
from jax.experimental import pallas as pl
from jax.experimental.pallas import tpu as pltpu
from jax.experimental.pallas import tpu_sc as plsc

CHUNK = 128
NCORES = 2
NSUB = 16
NTILES = NCORES * NSUB
NBUF = 4


def _mesh():
    return plsc.VectorSubcoreMesh(core_axis_name="c", subcore_axis_name="s")


def _tile_plan(wid, lo, t0):
    hi = lo + 4
    count = jnp.where(wid < t0, lo, hi)
    base = jnp.where(wid < t0, lo * wid, lo * t0 + hi * (wid - t0))
    return count, base


def _deg_kernel(n_pad, lo, t0, win, dw):
    rpt = n_pad // NSUB

    @functools.partial(
        pl.kernel,
        out_type=jax.ShapeDtypeStruct((NCORES, n_pad, dw), jnp.float32),
        mesh=_mesh(),
        compiler_params=pltpu.CompilerParams(use_tc_tiling_on_sc=False),
        scratch_types=[
            pltpu.VMEM((win, CHUNK), jnp.int32),
            pltpu.VMEM((CHUNK, dw), jnp.float32),
            pltpu.VMEM_SHARED((n_pad, dw), jnp.float32),
            pltpu.SemaphoreType.DMA,
        ],
    )
    def k(dst_hbm, zeros_hbm, ones_hbm, out_hbm, dst_v, ones_v, acc, ssem):
        c = lax.axis_index("c")
        s = lax.axis_index("s")
        wid = s * NCORES + c
        count, base = _tile_plan(wid, lo, t0)
        pltpu.sync_copy(dst_hbm.at[pl.ds(base, win)], dst_v)
        pltpu.sync_copy(ones_hbm, ones_v)
        pltpu.sync_copy(zeros_hbm.at[pl.ds(s * rpt, rpt)], acc.at[pl.ds(s * rpt, rpt)])
        plsc.subcore_barrier()

        def body(j, carry):
            pltpu.async_copy(ones_v, acc.at[dst_v.at[j]], ssem, add=True)
            return carry

        lax.fori_loop(0, count, body, 0)

        def drain(j, carry):
            pltpu.make_async_copy(ones_v, acc.at[dst_v.at[j]], ssem).wait()
            return carry

        lax.fori_loop(0, count, drain, 0)
        plsc.subcore_barrier()
        pltpu.sync_copy(acc.at[pl.ds(s * rpt, rpt)], out_hbm.at[c, pl.ds(s * rpt, rpt)])

    return k


def _scatter_kernel(n_pad, lo, t0, win, f):
    rpt = n_pad // NSUB

    @functools.partial(
        pl.kernel,
        out_type=jax.ShapeDtypeStruct((NCORES, n_pad, f), jnp.float32),
        mesh=_mesh(),
        compiler_params=pltpu.CompilerParams(use_tc_tiling_on_sc=False),
        scratch_types=[
            pltpu.VMEM((win, CHUNK), jnp.int32),
            pltpu.VMEM((win, CHUNK), jnp.int32),
            [pltpu.VMEM((CHUNK, f), jnp.float32) for _ in range(NBUF)],
            [pltpu.SemaphoreType.DMA for _ in range(NBUF)],
            [pltpu.SemaphoreType.DMA for _ in range(NBUF)],
            pltpu.VMEM_SHARED((n_pad, f), jnp.float32),
            pltpu.VMEM_SHARED((n_pad, f), jnp.float32),
        ],
    )
    def k(src_hbm, dst_hbm, y_hbm, zeros_hbm, out_hbm,
          src_v, dst_v, bufs, gsems, ssems, acc, y_sp):
        c = lax.axis_index("c")
        s = lax.axis_index("s")
        wid = s * NCORES + c
        count, base = _tile_plan(wid, lo, t0)
        pltpu.sync_copy(src_hbm.at[pl.ds(base, win)], src_v)
        pltpu.sync_copy(dst_hbm.at[pl.ds(base, win)], dst_v)
        pltpu.sync_copy(zeros_hbm.at[pl.ds(s * rpt, rpt)], acc.at[pl.ds(s * rpt, rpt)])
        pltpu.sync_copy(y_hbm.at[pl.ds(s * rpt, rpt)], y_sp.at[pl.ds(s * rpt, rpt)])
        plsc.subcore_barrier()

        def gather(j, b):
            pltpu.async_copy(y_sp.at[src_v.at[j]], bufs[b], gsems[b])

        def gather_wait(j, b):
            pltpu.make_async_copy(y_sp.at[src_v.at[j]], bufs[b], gsems[b]).wait()

        def scat(j, b):
            pltpu.async_copy(bufs[b], acc.at[dst_v.at[j]], ssems[b], add=True)

        def scat_wait(j, b):
            pltpu.make_async_copy(bufs[b], acc.at[dst_v.at[j]], ssems[b]).wait()

        for b in range(NBUF):
            gather(b, b)

        def body(i, carry):
            j0 = NBUF * i
            for b in range(NBUF):
                gather_wait(j0 + b, b)
                scat(j0 + b, b)
            for b in range(NBUF):
                scat_wait(j0 + b, b)
                gather(j0 + NBUF + b, b)
            return carry

        lax.fori_loop(0, count // NBUF - 1, body, 0)
        j0 = count - NBUF
        for b in range(NBUF):
            gather_wait(j0 + b, b)
            scat(j0 + b, b)
        for b in range(NBUF):
            scat_wait(j0 + b, b)

        plsc.subcore_barrier()
        pltpu.sync_copy(acc.at[pl.ds(s * rpt, rpt)], out_hbm.at[c, pl.ds(s * rpt, rpt)])

    return k


def _tc_prep(deg2, x, w1, n, n_pad):
    h = w1.shape[1]

    def body(deg_ref, x_ref, w_ref, y_ref, dinv_ref):
        d = deg_ref[0, :, 0:1] + deg_ref[1, :, 0:1] + 1.0
        dinv = lax.rsqrt(d)
        xt = jnp.dot(x_ref[...], w_ref[...], preferred_element_type=jnp.float32)
        xt_pad = jnp.concatenate(
            [xt, jnp.zeros((n_pad - n, h), jnp.float32)], axis=0)
        y_ref[...] = xt_pad * dinv
        dinv_ref[...] = dinv

    return pl.pallas_call(
        body,
        out_shape=[
            jax.ShapeDtypeStruct((n_pad, h), jnp.float32),
            jax.ShapeDtypeStruct((n_pad, 1), jnp.float32),
        ],
    )(deg2, x, w1)


def _tc_mid(s1, y1, dinv, b1, w2p, n_pad):
    w = w2p.shape[1]

    def body(s_ref, y_ref, dinv_ref, b_ref, w_ref, o_ref):
        o1 = (s_ref[0] + s_ref[1] + y_ref[...]) * dinv_ref[...] + b_ref[...]
        hid = jnp.maximum(o1, 0.0)
        o_ref[...] = jnp.dot(hid, w_ref[...], preferred_element_type=jnp.float32) * dinv_ref[...]

    return pl.pallas_call(
        body,
        out_shape=jax.ShapeDtypeStruct((n_pad, w), jnp.float32),
    )(s1, y1, dinv, b1, w2p)


def _tc_final(s2, y2, dinv, b2, n, c_out):
    def body(s_ref, y_ref, dinv_ref, b_ref, o_ref):
        o = (s_ref[0, :n, :] + s_ref[1, :n, :] + y_ref[:n, :]) * dinv_ref[:n, :]
        logits = o[:, 0:c_out] + b_ref[...]
        m = jnp.max(logits, axis=1, keepdims=True)
        ex = jnp.exp(logits - m)
        lse = jnp.log(jnp.sum(ex, axis=1, keepdims=True))
        o_ref[...] = logits - m - lse

    return pl.pallas_call(
        body,
        out_shape=jax.ShapeDtypeStruct((n, c_out), jnp.float32),
    )(s2, y2, dinv, b2)


def kernel(x, edge_index, W1, b1, W2, b2):
    n, _ = x.shape
    h = W1.shape[1]
    c_out = W2.shape[1]
    e = edge_index.shape[1]

    n_pad = -(-(n + 1) // (NSUB * 8)) * (NSUB * 8)
    dw = 8
    w2w = 8

    if e % (CHUNK * NBUF) == 0:
        src = edge_index[0].reshape(-1, CHUNK)
        dst = edge_index[1].reshape(-1, CHUNK)
    else:
        e_pad = -(-e // (CHUNK * NBUF)) * (CHUNK * NBUF)
        src = jnp.concatenate(
            [edge_index[0], jnp.zeros((e_pad - e,), jnp.int32)]).reshape(-1, CHUNK)
        dst = jnp.concatenate(
            [edge_index[1], jnp.full((e_pad - e,), n, jnp.int32)]).reshape(-1, CHUNK)
    n_chunks = src.shape[0]

    lo = (n_chunks // NTILES) // NBUF * NBUF
    n_hi = (n_chunks - NTILES * lo) // NBUF
    t0 = NTILES - n_hi
    win = lo + 4 if n_hi > 0 else lo

    zeros_dw = jnp.zeros((n_pad, dw), jnp.float32)
    ones_dw = jnp.ones((CHUNK, dw), jnp.float32)
    zeros_h = jnp.zeros((n_pad, h), jnp.float32)
    zeros_w2 = jnp.zeros((n_pad, w2w), jnp.float32)
    w2p = jnp.pad(W2, ((0, 0), (0, w2w - c_out)))

    deg2 = _deg_kernel(n_pad, lo, t0, win, dw)(dst, zeros_dw, ones_dw)
    y1, dinv = _tc_prep(deg2, x, W1, n, n_pad)
    s1 = _scatter_kernel(n_pad, lo, t0, win, h)(src, dst, y1, zeros_h)
    y2 = _tc_mid(s1, y1, dinv, b1.reshape(1, h), w2p, n_pad)
    s2 = _scatter_kernel(n_pad, lo, t0, win, w2w)(src, dst, y2, zeros_w2)
    return _tc_final(s2, y2, dinv, b2.reshape(1, c_out), n, c_out)


# speedup vs baseline: 1.7973x; 1.0026x over previous
import functools

import jax
import jax.numpy as jnp
from jax import lax
from jax.experimental import pallas as pl
from jax.experimental.pallas import tpu as pltpu
from jax.experimental.pallas import tpu_sc as plsc

CHUNK = 128
NCORES = 2
NSUB = 16
NTILES = NCORES * NSUB
NBUF = 4


def _mesh():
    return plsc.VectorSubcoreMesh(core_axis_name="c", subcore_axis_name="s")


def _tile_plan(wid, lo, t0):
    hi = lo + 4
    count = jnp.where(wid < t0, lo, hi)
    base = jnp.where(wid < t0, lo * wid, lo * t0 + hi * (wid - t0))
    return count, base


def _deg_kernel(n_pad, lo, t0, win, dw):
    rpt = n_pad // NSUB

    @functools.partial(
        pl.kernel,
        out_type=jax.ShapeDtypeStruct((NCORES, n_pad, dw), jnp.float32),
        mesh=_mesh(),
        compiler_params=pltpu.CompilerParams(use_tc_tiling_on_sc=False),
        scratch_types=[
            pltpu.VMEM((win, CHUNK), jnp.int32),
            pltpu.VMEM((CHUNK, dw), jnp.float32),
            pltpu.VMEM_SHARED((n_pad, dw), jnp.float32),
            pltpu.SemaphoreType.DMA,
        ],
    )
    def k(dst_hbm, zeros_hbm, ones_hbm, out_hbm, dst_v, ones_v, acc, ssem):
        c = lax.axis_index("c")
        s = lax.axis_index("s")
        wid = s * NCORES + c
        count, base = _tile_plan(wid, lo, t0)
        pltpu.sync_copy(dst_hbm.at[pl.ds(base, win)], dst_v)
        pltpu.sync_copy(ones_hbm, ones_v)
        pltpu.sync_copy(zeros_hbm.at[pl.ds(s * rpt, rpt)], acc.at[pl.ds(s * rpt, rpt)])
        plsc.subcore_barrier()

        def body(j, carry):
            pltpu.async_copy(ones_v, acc.at[dst_v.at[j]], ssem, add=True)
            return carry

        lax.fori_loop(0, count, body, 0)

        def drain(j, carry):
            pltpu.make_async_copy(ones_v, acc.at[dst_v.at[j]], ssem).wait()
            return carry

        lax.fori_loop(0, count, drain, 0)
        plsc.subcore_barrier()
        pltpu.sync_copy(acc.at[pl.ds(s * rpt, rpt)], out_hbm.at[c, pl.ds(s * rpt, rpt)])

    return k


def _scatter_kernel(n_pad, lo, t0, win, f):
    rpt = n_pad // NSUB

    @functools.partial(
        pl.kernel,
        out_type=jax.ShapeDtypeStruct((NCORES, n_pad, f), jnp.float32),
        mesh=_mesh(),
        compiler_params=pltpu.CompilerParams(use_tc_tiling_on_sc=False),
        scratch_types=[
            pltpu.VMEM((win, CHUNK), jnp.int32),
            pltpu.VMEM((win, CHUNK), jnp.int32),
            [pltpu.VMEM((CHUNK, f), jnp.float32) for _ in range(NBUF)],
            [pltpu.SemaphoreType.DMA for _ in range(NBUF)],
            [pltpu.SemaphoreType.DMA for _ in range(NBUF)],
            pltpu.VMEM_SHARED((n_pad, f), jnp.float32),
            pltpu.VMEM_SHARED((n_pad, f), jnp.float32),
        ],
    )
    def k(src_hbm, dst_hbm, y_hbm, zeros_hbm, out_hbm,
          src_v, dst_v, bufs, gsems, ssems, acc, y_sp):
        c = lax.axis_index("c")
        s = lax.axis_index("s")
        wid = s * NCORES + c
        count, base = _tile_plan(wid, lo, t0)
        pltpu.sync_copy(src_hbm.at[pl.ds(base, win)], src_v)
        pltpu.sync_copy(dst_hbm.at[pl.ds(base, win)], dst_v)
        pltpu.sync_copy(zeros_hbm.at[pl.ds(s * rpt, rpt)], acc.at[pl.ds(s * rpt, rpt)])
        pltpu.sync_copy(y_hbm.at[pl.ds(s * rpt, rpt)], y_sp.at[pl.ds(s * rpt, rpt)])
        plsc.subcore_barrier()

        def gather(j, b):
            pltpu.async_copy(y_sp.at[src_v.at[j]], bufs[b], gsems[b])

        def gather_wait(j, b):
            pltpu.make_async_copy(y_sp.at[src_v.at[j]], bufs[b], gsems[b]).wait()

        def scat(j, b):
            pltpu.async_copy(bufs[b], acc.at[dst_v.at[j]], ssems[b], add=True)

        def scat_wait(j, b):
            pltpu.make_async_copy(bufs[b], acc.at[dst_v.at[j]], ssems[b]).wait()

        for b in range(NBUF):
            gather(b, b)

        def body(i, carry):
            j0 = NBUF * i
            for b in range(NBUF):
                gather_wait(j0 + b, b)
                scat(j0 + b, b)
            for b in range(NBUF):
                scat_wait(j0 + b, b)
                gather(j0 + NBUF + b, b)
            return carry

        lax.fori_loop(0, count // NBUF - 1, body, 0)
        j0 = count - NBUF
        for b in range(NBUF):
            gather_wait(j0 + b, b)
            scat(j0 + b, b)
        for b in range(NBUF):
            scat_wait(j0 + b, b)

        plsc.subcore_barrier()
        pltpu.sync_copy(acc.at[pl.ds(s * rpt, rpt)], out_hbm.at[c, pl.ds(s * rpt, rpt)])

    return k


def _tc_prep(deg2, x, w1, n, n_pad):
    h = w1.shape[1]

    def body(deg_ref, x_ref, w_ref, y_ref, dinv_ref):
        d = deg_ref[0, :, 0:1] + deg_ref[1, :, 0:1] + 1.0
        dinv = lax.rsqrt(d)
        xt = jnp.dot(x_ref[...], w_ref[...], preferred_element_type=jnp.float32)
        xt_pad = jnp.concatenate(
            [xt, jnp.zeros((n_pad - n, h), jnp.float32)], axis=0)
        y_ref[...] = xt_pad * dinv
        dinv_ref[...] = dinv

    return pl.pallas_call(
        body,
        out_shape=[
            jax.ShapeDtypeStruct((n_pad, h), jnp.float32),
            jax.ShapeDtypeStruct((n_pad, 1), jnp.float32),
        ],
    )(deg2, x, w1)


def _tc_mid(s1, y1, dinv, b1, w2p, n_pad):
    w = w2p.shape[1]

    def body(s_ref, y_ref, dinv_ref, b_ref, w_ref, o_ref):
        o1 = (s_ref[0] + s_ref[1] + y_ref[...]) * dinv_ref[...] + b_ref[...]
        hid = jnp.maximum(o1, 0.0)
        o_ref[...] = jnp.dot(hid, w_ref[...], preferred_element_type=jnp.float32) * dinv_ref[...]

    return pl.pallas_call(
        body,
        out_shape=jax.ShapeDtypeStruct((n_pad, w), jnp.float32),
    )(s1, y1, dinv, b1, w2p)


def _tc_final(s2, y2, dinv, b2, n, c_out):
    def body(s_ref, y_ref, dinv_ref, b_ref, o_ref):
        o = (s_ref[0, :n, :] + s_ref[1, :n, :] + y_ref[:n, :]) * dinv_ref[:n, :]
        logits = o[:, 0:c_out] + b_ref[...]
        m = jnp.max(logits, axis=1, keepdims=True)
        ex = jnp.exp(logits - m)
        lse = jnp.log(jnp.sum(ex, axis=1, keepdims=True))
        o_ref[...] = logits - m - lse

    return pl.pallas_call(
        body,
        out_shape=jax.ShapeDtypeStruct((n, c_out), jnp.float32),
    )(s2, y2, dinv, b2)


def kernel(x, edge_index, W1, b1, W2, b2):
    n, _ = x.shape
    h = W1.shape[1]
    c_out = W2.shape[1]
    e = edge_index.shape[1]

    n_pad = -(-(n + 1) // (NSUB * 8)) * (NSUB * 8)
    dw = 8
    w2w = 8

    if e % (CHUNK * NBUF) == 0:
        (src,) = lax.optimization_barrier((edge_index[0].reshape(-1, CHUNK),))
        dst = edge_index[1].reshape(-1, CHUNK)
    else:
        e_pad = -(-e // (CHUNK * NBUF)) * (CHUNK * NBUF)
        src = jnp.concatenate(
            [edge_index[0], jnp.zeros((e_pad - e,), jnp.int32)]).reshape(-1, CHUNK)
        dst = jnp.concatenate(
            [edge_index[1], jnp.full((e_pad - e,), n, jnp.int32)]).reshape(-1, CHUNK)
    n_chunks = src.shape[0]

    lo = (n_chunks // NTILES) // NBUF * NBUF
    n_hi = (n_chunks - NTILES * lo) // NBUF
    t0 = NTILES - n_hi
    win = lo + 4 if n_hi > 0 else lo

    zeros_dw = jnp.zeros((n_pad, dw), jnp.float32)
    ones_dw = jnp.ones((CHUNK, dw), jnp.float32)
    zeros_h = jnp.zeros((n_pad, h), jnp.float32)
    zeros_w2 = jnp.zeros((n_pad, w2w), jnp.float32)
    w2p = jnp.pad(W2, ((0, 0), (0, w2w - c_out)))

    deg2 = _deg_kernel(n_pad, lo, t0, win, dw)(dst, zeros_dw, ones_dw)
    y1, dinv = _tc_prep(deg2, x, W1, n, n_pad)
    s1 = _scatter_kernel(n_pad, lo, t0, win, h)(src, dst, y1, zeros_h)
    y2 = _tc_mid(s1, y1, dinv, b1.reshape(1, h), w2p, n_pad)
    s2 = _scatter_kernel(n_pad, lo, t0, win, w2w)(src, dst, y2, zeros_w2)
    return _tc_final(s2, y2, dinv, b2.reshape(1, c_out), n, c_out)


# speedup vs baseline: 1.9031x; 1.0589x over previous
import functools

import jax
import jax.numpy as jnp
from jax import lax
from jax.experimental import pallas as pl
from jax.experimental.pallas import tpu as pltpu
from jax.experimental.pallas import tpu_sc as plsc

CHUNK = 128
NCORES = 2
NSUB = 16
NTILES = NCORES * NSUB
NBUF = 4


def _mesh():
    return plsc.VectorSubcoreMesh(core_axis_name="c", subcore_axis_name="s")


def _tile_plan(wid, lo, t0):
    hi = lo + 4
    count = jnp.where(wid < t0, lo, hi)
    base = jnp.where(wid < t0, lo * wid, lo * t0 + hi * (wid - t0))
    return count, base


def _deg_kernel(n_pad, lo, t0, win, dw):
    rpt = n_pad // NSUB

    @functools.partial(
        pl.kernel,
        out_type=jax.ShapeDtypeStruct((NCORES, n_pad), jnp.float32),
        mesh=_mesh(),
        compiler_params=pltpu.CompilerParams(
            use_tc_tiling_on_sc=False, needs_layout_passes=False),
        scratch_types=[
            pltpu.VMEM((win, CHUNK), jnp.int32),
            pltpu.VMEM((CHUNK, dw), jnp.float32),
            pltpu.VMEM_SHARED((n_pad, dw), jnp.float32),
            pltpu.VMEM((rpt, dw), jnp.float32),
            pltpu.VMEM((((rpt + 15) // 16) * 16,), jnp.float32),
            pltpu.SemaphoreType.DMA,
        ],
    )
    def k(dst_hbm, zeros_hbm, ones_hbm, out_hbm, dst_v, ones_v, acc, slab_v,
          col_v, ssem):
        c = lax.axis_index("c")
        s = lax.axis_index("s")
        wid = s * NCORES + c
        count, base = _tile_plan(wid, lo, t0)
        pltpu.sync_copy(dst_hbm.at[pl.ds(base, win)], dst_v)
        pltpu.sync_copy(ones_hbm, ones_v)
        pltpu.sync_copy(zeros_hbm.at[pl.ds(s * rpt, rpt)], acc.at[pl.ds(s * rpt, rpt)])
        plsc.subcore_barrier()

        def body(j, carry):
            pltpu.async_copy(ones_v, acc.at[dst_v.at[j]], ssem, add=True)
            return carry

        lax.fori_loop(0, count, body, 0)

        def drain(j, carry):
            pltpu.make_async_copy(ones_v, acc.at[dst_v.at[j]], ssem).wait()
            return carry

        lax.fori_loop(0, count, drain, 0)
        plsc.subcore_barrier()
        pltpu.sync_copy(acc.at[pl.ds(s * rpt, rpt)], slab_v)
        lanes = lax.iota(jnp.int32, 16)
        zeros16 = jnp.zeros((16,), jnp.int32)

        def extract(j, carry):
            rows = jnp.minimum(j * 16 + lanes, rpt - 1)
            col_v[pl.ds(j * 16, 16)] = plsc.load_gather(slab_v, [rows, zeros16])
            return carry

        lax.fori_loop(0, (rpt + 15) // 16, extract, 0)
        pltpu.sync_copy(col_v.at[pl.ds(0, rpt)], out_hbm.at[c, pl.ds(s * rpt, rpt)])

    return k


def _scatter_kernel(n_pad, lo, t0, win, f):
    rpt = n_pad // NSUB

    @functools.partial(
        pl.kernel,
        out_type=jax.ShapeDtypeStruct((NCORES, n_pad, f), jnp.float32),
        mesh=_mesh(),
        compiler_params=pltpu.CompilerParams(use_tc_tiling_on_sc=False),
        scratch_types=[
            pltpu.VMEM((win, CHUNK), jnp.int32),
            pltpu.VMEM((win, CHUNK), jnp.int32),
            [pltpu.VMEM((CHUNK, f), jnp.float32) for _ in range(NBUF)],
            [pltpu.SemaphoreType.DMA for _ in range(NBUF)],
            [pltpu.SemaphoreType.DMA for _ in range(NBUF)],
            pltpu.VMEM_SHARED((n_pad, f), jnp.float32),
            pltpu.VMEM_SHARED((n_pad, f), jnp.float32),
        ],
    )
    def k(src_hbm, dst_hbm, y_hbm, zeros_hbm, out_hbm,
          src_v, dst_v, bufs, gsems, ssems, acc, y_sp):
        c = lax.axis_index("c")
        s = lax.axis_index("s")
        wid = s * NCORES + c
        count, base = _tile_plan(wid, lo, t0)
        pltpu.sync_copy(src_hbm.at[pl.ds(base, win)], src_v)
        pltpu.sync_copy(dst_hbm.at[pl.ds(base, win)], dst_v)
        pltpu.sync_copy(zeros_hbm.at[pl.ds(s * rpt, rpt)], acc.at[pl.ds(s * rpt, rpt)])
        pltpu.sync_copy(y_hbm.at[pl.ds(s * rpt, rpt)], y_sp.at[pl.ds(s * rpt, rpt)])
        plsc.subcore_barrier()

        def gather(j, b):
            pltpu.async_copy(y_sp.at[src_v.at[j]], bufs[b], gsems[b])

        def gather_wait(j, b):
            pltpu.make_async_copy(y_sp.at[src_v.at[j]], bufs[b], gsems[b]).wait()

        def scat(j, b):
            pltpu.async_copy(bufs[b], acc.at[dst_v.at[j]], ssems[b], add=True)

        def scat_wait(j, b):
            pltpu.make_async_copy(bufs[b], acc.at[dst_v.at[j]], ssems[b]).wait()

        for b in range(NBUF):
            gather(b, b)

        def body(i, carry):
            j0 = NBUF * i
            for b in range(NBUF):
                gather_wait(j0 + b, b)
                scat(j0 + b, b)
            for b in range(NBUF):
                scat_wait(j0 + b, b)
                gather(j0 + NBUF + b, b)
            return carry

        lax.fori_loop(0, count // NBUF - 1, body, 0)
        j0 = count - NBUF
        for b in range(NBUF):
            gather_wait(j0 + b, b)
            scat(j0 + b, b)
        for b in range(NBUF):
            scat_wait(j0 + b, b)

        plsc.subcore_barrier()
        pltpu.sync_copy(acc.at[pl.ds(s * rpt, rpt)], out_hbm.at[c, pl.ds(s * rpt, rpt)])

    return k


def _tc_prep(deg2, x, w1, n, n_pad):
    h = w1.shape[1]

    def body(deg_ref, x_ref, w_ref, y_ref, dinv_ref):
        d = deg_ref[0, :] + deg_ref[1, :] + 1.0
        dinv = jnp.reshape(lax.rsqrt(d), (d.shape[0], 1))
        xt = jnp.dot(x_ref[...], w_ref[...], preferred_element_type=jnp.float32)
        xt_pad = jnp.concatenate(
            [xt, jnp.zeros((n_pad - n, h), jnp.float32)], axis=0)
        y_ref[...] = xt_pad * dinv
        dinv_ref[...] = dinv

    return pl.pallas_call(
        body,
        out_shape=[
            jax.ShapeDtypeStruct((n_pad, h), jnp.float32),
            jax.ShapeDtypeStruct((n_pad, 1), jnp.float32),
        ],
    )(deg2, x, w1)


def _tc_mid(s1, y1, dinv, b1, w2p, n_pad):
    w = w2p.shape[1]

    def body(s_ref, y_ref, dinv_ref, b_ref, w_ref, o_ref):
        o1 = (s_ref[0] + s_ref[1] + y_ref[...]) * dinv_ref[...] + b_ref[...]
        hid = jnp.maximum(o1, 0.0)
        o_ref[...] = jnp.dot(hid, w_ref[...], preferred_element_type=jnp.float32) * dinv_ref[...]

    return pl.pallas_call(
        body,
        out_shape=jax.ShapeDtypeStruct((n_pad, w), jnp.float32),
    )(s1, y1, dinv, b1, w2p)


def _tc_final(s2, y2, dinv, b2, n, c_out):
    def body(s_ref, y_ref, dinv_ref, b_ref, o_ref):
        o = (s_ref[0, :n, :] + s_ref[1, :n, :] + y_ref[:n, :]) * dinv_ref[:n, :]
        logits = o[:, 0:c_out] + b_ref[...]
        m = jnp.max(logits, axis=1, keepdims=True)
        ex = jnp.exp(logits - m)
        lse = jnp.log(jnp.sum(ex, axis=1, keepdims=True))
        o_ref[...] = logits - m - lse

    return pl.pallas_call(
        body,
        out_shape=jax.ShapeDtypeStruct((n, c_out), jnp.float32),
    )(s2, y2, dinv, b2)


def kernel(x, edge_index, W1, b1, W2, b2):
    n, _ = x.shape
    h = W1.shape[1]
    c_out = W2.shape[1]
    e = edge_index.shape[1]

    n_pad = -(-(n + 1) // (NSUB * 8)) * (NSUB * 8)
    dw = 8
    w2w = 8

    if e % (CHUNK * NBUF) == 0:
        (src,) = lax.optimization_barrier((edge_index[0].reshape(-1, CHUNK),))
        dst = edge_index[1].reshape(-1, CHUNK)
    else:
        e_pad = -(-e // (CHUNK * NBUF)) * (CHUNK * NBUF)
        src = jnp.concatenate(
            [edge_index[0], jnp.zeros((e_pad - e,), jnp.int32)]).reshape(-1, CHUNK)
        dst = jnp.concatenate(
            [edge_index[1], jnp.full((e_pad - e,), n, jnp.int32)]).reshape(-1, CHUNK)
    n_chunks = src.shape[0]

    lo = (n_chunks // NTILES) // NBUF * NBUF
    n_hi = (n_chunks - NTILES * lo) // NBUF
    t0 = NTILES - n_hi
    win = lo + 4 if n_hi > 0 else lo

    zeros_dw = jnp.zeros((n_pad, dw), jnp.float32)
    ones_dw = jnp.ones((CHUNK, dw), jnp.float32)
    zeros_h = jnp.zeros((n_pad, h), jnp.float32)
    zeros_w2 = jnp.zeros((n_pad, w2w), jnp.float32)
    w2p = jnp.pad(W2, ((0, 0), (0, w2w - c_out)))

    deg2 = _deg_kernel(n_pad, lo, t0, win, dw)(dst, zeros_dw, ones_dw)
    y1, dinv = _tc_prep(deg2, x, W1, n, n_pad)
    s1 = _scatter_kernel(n_pad, lo, t0, win, h)(src, dst, y1, zeros_h)
    y2 = _tc_mid(s1, y1, dinv, b1.reshape(1, h), w2p, n_pad)
    s2 = _scatter_kernel(n_pad, lo, t0, win, w2w)(src, dst, y2, zeros_w2)
    return _tc_final(s2, y2, dinv, b2.reshape(1, c_out), n, c_out)


# speedup vs baseline: 1.9057x; 1.0014x over previous
import functools

import jax
import jax.numpy as jnp
from jax import lax
from jax.experimental import pallas as pl
from jax.experimental.pallas import tpu as pltpu
from jax.experimental.pallas import tpu_sc as plsc

CHUNK = 128
NCORES = 2
NSUB = 16
NTILES = NCORES * NSUB
NBUF = 4


def _mesh():
    return plsc.VectorSubcoreMesh(core_axis_name="c", subcore_axis_name="s")


def _tile_plan(wid, lo, t0):
    hi = lo + 4
    count = jnp.where(wid < t0, lo, hi)
    base = jnp.where(wid < t0, lo * wid, lo * t0 + hi * (wid - t0))
    return count, base


def _deg_kernel(n_pad, lo, t0, win, dw):
    rpt = n_pad // NSUB

    @functools.partial(
        pl.kernel,
        out_type=jax.ShapeDtypeStruct((NCORES, n_pad), jnp.float32),
        mesh=_mesh(),
        compiler_params=pltpu.CompilerParams(
            use_tc_tiling_on_sc=False, needs_layout_passes=False),
        scratch_types=[
            pltpu.VMEM((win, CHUNK), jnp.int32),
            pltpu.VMEM((CHUNK, dw), jnp.float32),
            pltpu.VMEM_SHARED((n_pad, dw), jnp.float32),
            pltpu.VMEM((rpt, dw), jnp.float32),
            pltpu.VMEM((((rpt + 15) // 16) * 16,), jnp.float32),
            pltpu.SemaphoreType.DMA,
        ],
    )
    def k(dst_hbm, zeros_hbm, ones_hbm, out_hbm, dst_v, ones_v, acc, slab_v,
          col_v, ssem):
        c = lax.axis_index("c")
        s = lax.axis_index("s")
        wid = s * NCORES + c
        count, base = _tile_plan(wid, lo, t0)
        pltpu.sync_copy(dst_hbm.at[pl.ds(base, win)], dst_v)
        pltpu.sync_copy(ones_hbm, ones_v)
        pltpu.sync_copy(zeros_hbm.at[pl.ds(s * rpt, rpt)], acc.at[pl.ds(s * rpt, rpt)])
        plsc.subcore_barrier()

        def body(j, carry):
            pltpu.async_copy(ones_v, acc.at[dst_v.at[j]], ssem, add=True)
            return carry

        lax.fori_loop(0, count, body, 0)

        def drain(j, carry):
            pltpu.make_async_copy(ones_v, acc.at[dst_v.at[j]], ssem).wait()
            return carry

        lax.fori_loop(0, count, drain, 0)
        plsc.subcore_barrier()
        pltpu.sync_copy(acc.at[pl.ds(s * rpt, rpt)], slab_v)
        lanes = lax.iota(jnp.int32, 16)
        zeros16 = jnp.zeros((16,), jnp.int32)

        def extract(j, carry):
            rows = jnp.minimum(j * 16 + lanes, rpt - 1)
            col_v[pl.ds(j * 16, 16)] = plsc.load_gather(slab_v, [rows, zeros16])
            return carry

        lax.fori_loop(0, (rpt + 15) // 16, extract, 0)
        pltpu.sync_copy(col_v.at[pl.ds(0, rpt)], out_hbm.at[c, pl.ds(s * rpt, rpt)])

    return k


def _scatter_kernel(n_pad, lo, t0, win, f):
    rpt = n_pad // NSUB

    @functools.partial(
        pl.kernel,
        out_type=jax.ShapeDtypeStruct((NCORES, n_pad, f), jnp.float32),
        mesh=_mesh(),
        compiler_params=pltpu.CompilerParams(use_tc_tiling_on_sc=False),
        scratch_types=[
            pltpu.VMEM((win, CHUNK), jnp.int32),
            pltpu.VMEM((win, CHUNK), jnp.int32),
            [pltpu.VMEM((CHUNK, f), jnp.float32) for _ in range(NBUF)],
            [pltpu.SemaphoreType.DMA for _ in range(NBUF)],
            [pltpu.SemaphoreType.DMA for _ in range(NBUF)],
            pltpu.VMEM_SHARED((n_pad, f), jnp.float32),
            pltpu.VMEM_SHARED((n_pad, f), jnp.float32),
        ],
    )
    def k(src_hbm, dst_hbm, y_hbm, zeros_hbm, out_hbm,
          src_v, dst_v, bufs, gsems, ssems, acc, y_sp):
        c = lax.axis_index("c")
        s = lax.axis_index("s")
        wid = s * NCORES + c
        count, base = _tile_plan(wid, lo, t0)
        pltpu.sync_copy(src_hbm.at[pl.ds(base, win)], src_v)
        pltpu.sync_copy(dst_hbm.at[pl.ds(base, win)], dst_v)
        pltpu.sync_copy(zeros_hbm.at[pl.ds(s * rpt, rpt)], acc.at[pl.ds(s * rpt, rpt)])
        pltpu.sync_copy(y_hbm.at[pl.ds(s * rpt, rpt)], y_sp.at[pl.ds(s * rpt, rpt)])
        plsc.subcore_barrier()

        def gather(j, b):
            pltpu.async_copy(y_sp.at[src_v.at[j]], bufs[b], gsems[b])

        def gather_wait(j, b):
            pltpu.make_async_copy(y_sp.at[src_v.at[j]], bufs[b], gsems[b]).wait()

        def scat(j, b):
            pltpu.async_copy(bufs[b], acc.at[dst_v.at[j]], ssems[b], add=True)

        def scat_wait(j, b):
            pltpu.make_async_copy(bufs[b], acc.at[dst_v.at[j]], ssems[b]).wait()

        for b in range(NBUF):
            gather(b, b)

        def body(i, carry):
            j0 = NBUF * i
            for b in range(NBUF):
                gather_wait(j0 + b, b)
                scat(j0 + b, b)
            for b in range(NBUF):
                scat_wait(j0 + b, b)
                gather(j0 + NBUF + b, b)
            return carry

        lax.fori_loop(0, count // NBUF - 1, body, 0)
        j0 = count - NBUF
        for b in range(NBUF):
            gather_wait(j0 + b, b)
            scat(j0 + b, b)
        for b in range(NBUF):
            scat_wait(j0 + b, b)

        plsc.subcore_barrier()
        pltpu.sync_copy(acc.at[pl.ds(s * rpt, rpt)], out_hbm.at[c, pl.ds(s * rpt, rpt)])

    return k


def _tc_prep(deg2, x, w1, n, n_pad):
    h = w1.shape[1]

    def body(deg_ref, x_ref, w_ref, y_ref, dinv_ref):
        d = deg_ref[0, :] + deg_ref[1, :] + 1.0
        dinv = jnp.reshape(lax.rsqrt(d), (d.shape[0], 1))
        xt = jnp.dot(x_ref[...], w_ref[...], preferred_element_type=jnp.float32)
        xt_pad = jnp.concatenate(
            [xt, jnp.zeros((n_pad - n, h), jnp.float32)], axis=0)
        y_ref[...] = xt_pad * dinv
        dinv_ref[...] = dinv

    return pl.pallas_call(
        body,
        out_shape=[
            jax.ShapeDtypeStruct((n_pad, h), jnp.float32),
            jax.ShapeDtypeStruct((n_pad, 1), jnp.float32),
        ],
    )(deg2, x, w1)


def _tc_mid(s1, y1, dinv, b1, w2p, n_pad):
    w = w2p.shape[1]

    def body(s_ref, y_ref, dinv_ref, b_ref, w_ref, o_ref):
        o1 = (s_ref[0] + s_ref[1] + y_ref[...]) * dinv_ref[...] + b_ref[...]
        hid = jnp.maximum(o1, 0.0)
        o_ref[...] = jnp.dot(hid, w_ref[...], preferred_element_type=jnp.float32) * dinv_ref[...]

    return pl.pallas_call(
        body,
        out_shape=jax.ShapeDtypeStruct((n_pad, w), jnp.float32),
    )(s1, y1, dinv, b1, w2p)


def _tc_final(s2, y2, dinv, b2, n, c_out):
    def body(s_ref, y_ref, dinv_ref, b_ref, o_ref):
        o = (s_ref[0, :n, :] + s_ref[1, :n, :] + y_ref[:n, :]) * dinv_ref[:n, :]
        logits = o[:, 0:c_out] + b_ref[...]
        m = jnp.max(logits, axis=1, keepdims=True)
        ex = jnp.exp(logits - m)
        lse = jnp.log(jnp.sum(ex, axis=1, keepdims=True))
        o_ref[...] = logits - m - lse

    return pl.pallas_call(
        body,
        out_shape=jax.ShapeDtypeStruct((n, c_out), jnp.float32),
    )(s2, y2, dinv, b2)


def kernel(x, edge_index, W1, b1, W2, b2):
    n, _ = x.shape
    h = W1.shape[1]
    c_out = W2.shape[1]
    e = edge_index.shape[1]

    n_pad = -(-(n + 1) // (NSUB * 8)) * (NSUB * 8)
    dw = 8
    w2w = 8

    if e % (CHUNK * NBUF) == 0:
        (src,) = lax.optimization_barrier((edge_index[0].reshape(-1, CHUNK),))
        (dst,) = lax.optimization_barrier((edge_index[1].reshape(-1, CHUNK),))
    else:
        e_pad = -(-e // (CHUNK * NBUF)) * (CHUNK * NBUF)
        src = jnp.concatenate(
            [edge_index[0], jnp.zeros((e_pad - e,), jnp.int32)]).reshape(-1, CHUNK)
        dst = jnp.concatenate(
            [edge_index[1], jnp.full((e_pad - e,), n, jnp.int32)]).reshape(-1, CHUNK)
    n_chunks = src.shape[0]

    lo = (n_chunks // NTILES) // NBUF * NBUF
    n_hi = (n_chunks - NTILES * lo) // NBUF
    t0 = NTILES - n_hi
    win = lo + 4 if n_hi > 0 else lo

    zeros_dw = jnp.zeros((n_pad, dw), jnp.float32)
    ones_dw = jnp.ones((CHUNK, dw), jnp.float32)
    zeros_h = jnp.zeros((n_pad, h), jnp.float32)
    zeros_w2 = jnp.zeros((n_pad, w2w), jnp.float32)
    w2p = jnp.pad(W2, ((0, 0), (0, w2w - c_out)))

    deg2 = _deg_kernel(n_pad, lo, t0, win, dw)(dst, zeros_dw, ones_dw)
    y1, dinv = _tc_prep(deg2, x, W1, n, n_pad)
    s1 = _scatter_kernel(n_pad, lo, t0, win, h)(src, dst, y1, zeros_h)
    y2 = _tc_mid(s1, y1, dinv, b1.reshape(1, h), w2p, n_pad)
    s2 = _scatter_kernel(n_pad, lo, t0, win, w2w)(src, dst, y2, zeros_w2)
    return _tc_final(s2, y2, dinv, b2.reshape(1, c_out), n, c_out)
